# baseline (speedup 1.0000x reference)
import functools

import numpy as np
import jax
import jax.numpy as jnp
from jax import lax
from jax.experimental import pallas as pl
from jax.experimental.pallas import tpu as pltpu
from jax.experimental.pallas import tpu_sc as plsc

_B = 32
_N = 128
_C = 64
_P = _N * (_N + 1) // 2
_NC, _NS = 2, 16
_CG = 4
_NCG = _C // _CG
_IQ = 32
_NQ = _N // _IQ
_STEP = 3

_ti, _tj = np.tril_indices(_N)
_TABLE = (((_ti % _IQ) << 7) | _tj).astype(np.int32)
_CHUNKS = [(0, 33), (33, 132), (130, 292), (291, 516)]
assert all((_k1 - _k0) % _STEP == 0 for _k0, _k1 in _CHUNKS)


@functools.partial(
    pl.kernel,
    out_type=jax.ShapeDtypeStruct((_B * _C, _P), jnp.float32),
    mesh=plsc.VectorSubcoreMesh(core_axis_name="c", subcore_axis_name="s"),
    compiler_params=pltpu.CompilerParams(needs_layout_passes=False),
    scratch_types=[
        pltpu.VMEM((_P,), jnp.int32),
        pltpu.VMEM((_IQ, _CG, _N), jnp.float32),
        pltpu.VMEM((_IQ, _CG, _N), jnp.float32),
    ] + [pltpu.VMEM((_P,), jnp.float32) for _ in range(2 * _CG)] + [
        pltpu.SemaphoreType.DMA,
        pltpu.SemaphoreType.DMA,
        pltpu.SemaphoreType.DMA,
        pltpu.SemaphoreType.DMA,
    ],
)
def _tril_compact(xt_hbm, table_hbm, out_hbm, table_v, slab0, slab1, *rest):
    rows = (rest[:_CG], rest[_CG:2 * _CG])
    gsems = (rest[2 * _CG], rest[2 * _CG + 1])
    wsems = (rest[2 * _CG + 2], rest[2 * _CG + 3])
    slabs = (slab0, slab1)
    w = lax.axis_index("s") * _NC + lax.axis_index("c")
    pltpu.sync_copy(table_hbm, table_v)

    def stream(cg, q, sb):
        return pltpu.make_async_copy(
            xt_hbm.at[w, pl.ds(q * _IQ, _IQ), pl.ds(cg * _CG, _CG), :],
            slabs[sb], gsems[sb])

    def writeback(cg, par, cc):
        return pltpu.make_async_copy(
            rows[par][cc], out_hbm.at[w * _C + cg * _CG + cc], wsems[par])

    def compact(q, sb, par):
        k0, k1 = _CHUNKS[q]
        for cc in range(_CG):
            idx_c = jnp.full((16,), cc, jnp.int32)
            buf = rows[par][cc]

            @plsc.parallel_loop(k0, k1, step=_STEP)
            def _chunk(k, idx_c=idx_c, buf=buf, sb=sb):
                for u in range(_STEP):
                    t = table_v[pl.ds((k + u) * 16, 16)]
                    vals = plsc.load_gather(
                        slabs[sb],
                        [lax.shift_right_logical(t, 7), idx_c,
                         lax.bitwise_and(t, 127)])
                    buf[pl.ds((k + u) * 16, 16)] = vals

    def group(cg, par):
        for q in range(_NQ):
            sb = q % 2
            stream(cg, q, sb).wait()
            nq, ncg = (q + 1, cg) if q + 1 < _NQ else (0, lax.rem(cg + 1, _NCG))
            stream(ncg, nq, 1 - sb).start()
            pass
            compact(q, sb, par)
        pass

    stream(0, 0, 0).start()
    pass

    def pair(g, _):
        group(2 * g, 0)
        group(2 * g + 1, 1)
        return ()

    lax.fori_loop(0, _NCG // 2, pair, ())

    stream(0, 0, 0).wait()
    pass


def kernel(x):
    xt = jnp.transpose(x, (0, 1, 3, 2))
    out2d = _tril_compact(xt, jnp.asarray(_TABLE))
    return jnp.transpose(out2d.reshape(_B, _C, _P), (0, 2, 1))


# speedup vs baseline: 1.1867x; 1.1867x over previous
import functools

import numpy as np
import jax
import jax.numpy as jnp
from jax import lax
from jax.experimental import pallas as pl
from jax.experimental.pallas import tpu as pltpu
from jax.experimental.pallas import tpu_sc as plsc

_B = 32
_N = 128
_C = 64
_P = _N * (_N + 1) // 2
_NC, _NS = 2, 16
_CG = 8
_NCG = _C // _CG
_IQ = 32
_NQ = _N // _IQ
_SEG = 4096

_ti, _tj = np.tril_indices(_N)
_TABLE = (((_ti % _IQ) << 7) | _tj).astype(np.int32)
_RANGES = [
    [(0, 33, 0)],
    [(33, 130, 0)],
    [(130, 256, 0), (256, 291, 4096)],
    [(291, 512, 4096)],
]


@functools.partial(
    pl.kernel,
    out_type=jax.ShapeDtypeStruct((_B * _C, _P), jnp.float32),
    mesh=plsc.VectorSubcoreMesh(core_axis_name="c", subcore_axis_name="s"),
    compiler_params=pltpu.CompilerParams(needs_layout_passes=False),
    scratch_types=[
        pltpu.VMEM((_P,), jnp.int32),
        pltpu.VMEM((_IQ, _CG, _N), jnp.float32),
        pltpu.VMEM((_IQ, _CG, _N), jnp.float32),
    ] + [pltpu.VMEM((_SEG,), jnp.float32) for _ in range(_CG)] + [
        pltpu.SemaphoreType.DMA,
        pltpu.SemaphoreType.DMA,
        pltpu.SemaphoreType.DMA,
        pltpu.SemaphoreType.DMA,
    ],
)
def _tril_compact(xt_hbm, table_hbm, out_hbm, table_v, slab0, slab1, *rest):
    rows = rest[:_CG]
    gsems = (rest[_CG], rest[_CG + 1])
    slabs = (slab0, slab1)
    w = lax.axis_index("s") * _NC + lax.axis_index("c")
    pltpu.sync_copy(table_hbm, table_v)

    def stream(cg, q, sb):
        return pltpu.make_async_copy(
            xt_hbm.at[w, pl.ds(q * _IQ, _IQ), pl.ds(cg * _CG, _CG), :],
            slabs[sb], gsems[sb])

    def writeback(cg, cc, seg):
        return pltpu.make_async_copy(
            rows[cc],
            out_hbm.at[w * _C + cg * _CG + cc, pl.ds(seg * _SEG, _SEG)],
            rest[_CG + 2 + seg])

    idx_cs = [jnp.full((16,), cc, jnp.int32) for cc in range(_CG)]

    def compact_range(k0, k1, base, sb):
        @plsc.parallel_loop(k0, k1)
        def _chunk(k, sb=sb):
            t = table_v[pl.ds(k * 16, 16)]
            t_i = lax.shift_right_logical(t, 7)
            t_j = lax.bitwise_and(t, 127)
            for cc in range(_CG):
                vals = plsc.load_gather(slabs[sb], [t_i, idx_cs[cc], t_j])
                rows[cc][pl.ds(k * 16 - base, 16)] = vals

    def group(cg):
        for q in range(_NQ):
            sb = q % 2
            stream(cg, q, sb).wait()
            nq, ncg = (q + 1, cg) if q + 1 < _NQ else (0, lax.rem(cg + 1, _NCG))
            stream(ncg, nq, 1 - sb).start()
            if q == 0:
                for cc in range(_CG):
                    writeback(cg, cc, 1).wait()
            if q != 2:
                for (k0, k1, base) in _RANGES[q]:
                    compact_range(k0, k1, base, sb)
                if q == 3:
                    for cc in range(_CG):
                        writeback(cg, cc, 1).start()
            else:
                k0, k1, base = _RANGES[2][0]
                compact_range(k0, k1, base, sb)
                for cc in range(_CG):
                    writeback(cg, cc, 0).start()
                for cc in range(_CG):
                    writeback(cg, cc, 0).wait()
                k0, k1, base = _RANGES[2][1]
                compact_range(k0, k1, base, sb)

    stream(0, 0, 0).start()
    for cc in range(_CG):
        writeback(0, cc, 1).start()

    def body(g, _):
        group(g)
        return ()

    lax.fori_loop(0, _NCG, body, ())

    stream(0, 0, 0).wait()
    for cc in range(_CG):
        writeback(_NCG - 1, cc, 1).wait()


def kernel(x):
    xt = jnp.transpose(x, (0, 1, 3, 2))
    out2d = _tril_compact(xt, jnp.asarray(_TABLE))
    tail = xt[:, _N - 1, :, _N // 2:].reshape(_B * _C, _N // 2)
    out2d = lax.dynamic_update_slice(out2d, tail, (0, 2 * _SEG))
    return jnp.transpose(out2d.reshape(_B, _C, _P), (0, 2, 1))
